# Initial kernel scaffold; baseline (speedup 1.0000x reference)
import jax
import jax.numpy as jnp
from jax.experimental import pallas as pl


def kernel(x, adj, W1, b1, W2, b2):
    raise NotImplementedError("write your pallas kernel here")


# speedup vs baseline: 1.0376x; 1.0376x over previous
import jax
import jax.numpy as jnp
from jax.experimental import pallas as pl
from jax.experimental.pallas import tpu as pltpu

_BM = 400


def _selu(v):
    alpha = 1.6732632423543772
    scale = 1.0507009873554805
    return scale * jnp.where(v > 0.0, v, alpha * (jnp.exp(jnp.minimum(v, 0.0)) - 1.0))


def _layer1_body(adj_ref, x_ref, w1_ref, b1_ref, w2_ref, s2_ref):
    t = jnp.dot(adj_ref[...], x_ref[...], preferred_element_type=jnp.float32)
    h = _selu(jnp.dot(t, w1_ref[...], preferred_element_type=jnp.float32)
              + b1_ref[...])
    s2_ref[...] = jnp.dot(h, w2_ref[...], preferred_element_type=jnp.float32)


def _layer2_body(adj_ref, s2_ref, b2_ref, out_ref):
    t = jnp.dot(adj_ref[...], s2_ref[...], preferred_element_type=jnp.float32)
    out_ref[...] = _selu(t + b2_ref[...])


def kernel(x, adj, W1, b1, W2, b2):
    n, f_in = x.shape
    f_hid = W1.shape[1]
    f_out = W2.shape[1]
    grid = (pl.cdiv(n, _BM),)
    b1r = b1.reshape(1, f_hid)
    b2r = b2.reshape(1, f_out)

    s2 = pl.pallas_call(
        _layer1_body,
        grid=grid,
        in_specs=[
            pl.BlockSpec((_BM, n), lambda i: (i, 0)),
            pl.BlockSpec((n, f_in), lambda i: (0, 0)),
            pl.BlockSpec((f_in, f_hid), lambda i: (0, 0)),
            pl.BlockSpec((1, f_hid), lambda i: (0, 0)),
            pl.BlockSpec((f_hid, f_out), lambda i: (0, 0)),
        ],
        out_specs=pl.BlockSpec((_BM, f_out), lambda i: (i, 0)),
        out_shape=jax.ShapeDtypeStruct((n, f_out), jnp.float32),
        compiler_params=pltpu.CompilerParams(
            dimension_semantics=("arbitrary",),
        ),
    )(adj, x, W1, b1r, W2)

    out = pl.pallas_call(
        _layer2_body,
        grid=grid,
        in_specs=[
            pl.BlockSpec((_BM, n), lambda i: (i, 0)),
            pl.BlockSpec((n, f_out), lambda i: (0, 0)),
            pl.BlockSpec((1, f_out), lambda i: (0, 0)),
        ],
        out_specs=pl.BlockSpec((_BM, f_out), lambda i: (i, 0)),
        out_shape=jax.ShapeDtypeStruct((n, f_out), jnp.float32),
        compiler_params=pltpu.CompilerParams(
            dimension_semantics=("arbitrary",),
        ),
    )(adj, s2, b2r)

    return out


# speedup vs baseline: 1.0704x; 1.0315x over previous
import jax
import jax.numpy as jnp
from jax.experimental import pallas as pl
from jax.experimental.pallas import tpu as pltpu

_BM = 400


def _selu(v):
    alpha = 1.6732632423543772
    scale = 1.0507009873554805
    return scale * jnp.where(v > 0.0, v, alpha * (jnp.exp(jnp.minimum(v, 0.0)) - 1.0))


def _fused_body(nb, adj_ref, x_ref, w1_ref, b1_ref, w2_ref, b2_ref,
                out_ref, s2_ref):
    i = pl.program_id(0)

    @pl.when(i < nb)
    def _layer1():
        t = jnp.dot(adj_ref[...], x_ref[...],
                    preferred_element_type=jnp.float32)
        h = _selu(jnp.dot(t, w1_ref[...], preferred_element_type=jnp.float32)
                  + b1_ref[...])
        s2_ref[pl.ds(i * _BM, _BM), :] = jnp.dot(
            h, w2_ref[...], preferred_element_type=jnp.float32)

    @pl.when(i >= nb)
    def _layer2():
        t = jnp.dot(adj_ref[...], s2_ref[...],
                    preferred_element_type=jnp.float32)
        out_ref[...] = _selu(t + b2_ref[...])


def kernel(x, adj, W1, b1, W2, b2):
    n, f_in = x.shape
    f_hid = W1.shape[1]
    f_out = W2.shape[1]
    nb = n // _BM
    b1r = b1.reshape(1, f_hid)
    b2r = b2.reshape(1, f_out)

    body = lambda *refs: _fused_body(nb, *refs)

    out = pl.pallas_call(
        body,
        grid=(2 * nb,),
        in_specs=[
            pl.BlockSpec((_BM, n), lambda i: (jax.lax.rem(i, nb), 0)),
            pl.BlockSpec((n, f_in), lambda i: (0, 0)),
            pl.BlockSpec((f_in, f_hid), lambda i: (0, 0)),
            pl.BlockSpec((1, f_hid), lambda i: (0, 0)),
            pl.BlockSpec((f_hid, f_out), lambda i: (0, 0)),
            pl.BlockSpec((1, f_out), lambda i: (0, 0)),
        ],
        out_specs=pl.BlockSpec(
            (_BM, f_out),
            lambda i: (jnp.maximum(i - nb, 0), 0)),
        out_shape=jax.ShapeDtypeStruct((n, f_out), jnp.float32),
        scratch_shapes=[pltpu.VMEM((n, f_out), jnp.float32)],
        compiler_params=pltpu.CompilerParams(
            dimension_semantics=("arbitrary",),
        ),
    )(adj, x, W1, b1r, W2, b2r)

    return out
